# Initial kernel scaffold; baseline (speedup 1.0000x reference)
import functools

import jax
import jax.numpy as jnp
from jax import lax
from jax.experimental import pallas as pl
from jax.experimental.pallas import tpu as pltpu
from jax.experimental.pallas import tpu_sc as plsc

B = 16
S = 4096
D = 1024
R = 2048
OUT_S = R + 1
NC = 2
NS = 16
NW = NC * NS
RPW = (B * R) // NW
CH = 32
NCH = RPW // CH
L = 16


def _make_kernel():
    mesh = plsc.VectorSubcoreMesh(core_axis_name="c", subcore_axis_name="s")

    @functools.partial(
        pl.kernel,
        mesh=mesh,
        out_type=jax.ShapeDtypeStruct((B, OUT_S, D), jnp.float32),
        scratch_types=[
            pltpu.VMEM((RPW,), jnp.int32),
            pltpu.VMEM((CH, D), jnp.float32),
            pltpu.SemaphoreType.DMA,
        ],
    )
    def gather_kernel(table_hbm, idx_hbm, out_hbm, idx_v, buf_v, sem):
        wid = lax.axis_index("s") * NC + lax.axis_index("c")
        b = wid // 2
        h = wid % 2

        pltpu.sync_copy(idx_hbm.at[pl.ds(wid * RPW, RPW)], idx_v)

        off = b * S + 1
        for j in range(RPW // L):
            idx_v[pl.ds(j * L, L)] = idx_v[pl.ds(j * L, L)] + off

        @pl.when(h == 0)
        def _():
            pltpu.sync_copy(table_hbm.at[pl.ds(b * S, 1)], buf_v.at[pl.ds(0, 1)])
            pltpu.sync_copy(buf_v.at[pl.ds(0, 1)], out_hbm.at[b, pl.ds(0, 1)])

        row0 = 1 + h * RPW

        def chunk(c, carry):
            pltpu.async_copy(
                table_hbm.at[idx_v.at[pl.ds(c * CH, CH)]], buf_v, sem
            ).wait()
            pltpu.sync_copy(buf_v, out_hbm.at[b, pl.ds(row0 + c * CH, CH)])
            return carry

        lax.fori_loop(0, NCH, chunk, 0)

    return gather_kernel


_GATHER = _make_kernel()


def kernel(data, remain_idx):
    table = data.reshape(B * S, D)
    idx_flat = remain_idx.reshape(B * R).astype(jnp.int32)
    return _GATHER(table, idx_flat)


# speedup vs baseline: 1.9733x; 1.9733x over previous
import functools

import jax
import jax.numpy as jnp
from jax import lax
from jax.experimental import pallas as pl
from jax.experimental.pallas import tpu as pltpu
from jax.experimental.pallas import tpu_sc as plsc

B = 16
S = 4096
D = 1024
R = 2048
OUT_S = R + 1
NROWS = B * OUT_S
NC = 2
NS = 16
NW = NC * NS
RPW = 1024
XTRA = 8
CH = 32
NCH = RPW // CH


def _make_kernel():
    mesh = plsc.VectorSubcoreMesh(core_axis_name="c", subcore_axis_name="s")

    @functools.partial(
        pl.kernel,
        mesh=mesh,
        out_type=jax.ShapeDtypeStruct((NROWS, D), jnp.float32),
        scratch_types=[
            pltpu.VMEM((RPW + XTRA,), jnp.int32),
            pltpu.VMEM((CH, D), jnp.float32),
            pltpu.SemaphoreType.DMA,
        ],
    )
    def gather_kernel(table_hbm, idx_hbm, out_hbm, idx_v, buf_v, sem):
        wid = lax.axis_index("s") * NC + lax.axis_index("c")
        base = wid * RPW

        pltpu.sync_copy(idx_hbm.at[pl.ds(base, RPW)], idx_v.at[pl.ds(0, RPW)])

        def chunk(c, carry):
            pltpu.async_copy(
                table_hbm.at[idx_v.at[pl.ds(c * CH, CH)]], buf_v, sem
            ).wait()
            pltpu.sync_copy(buf_v, out_hbm.at[pl.ds(base + c * CH, CH)])
            return carry

        lax.fori_loop(0, NCH, chunk, 0)

        @pl.when(wid < 2)
        def _():
            xbase = NW * RPW + wid * XTRA
            pltpu.sync_copy(
                idx_hbm.at[pl.ds(xbase, XTRA)], idx_v.at[pl.ds(RPW, XTRA)]
            )
            pltpu.async_copy(
                table_hbm.at[idx_v.at[pl.ds(RPW, XTRA)]],
                buf_v.at[pl.ds(0, XTRA)],
                sem,
            ).wait()
            pltpu.sync_copy(
                buf_v.at[pl.ds(0, XTRA)], out_hbm.at[pl.ds(xbase, XTRA)]
            )

    return gather_kernel


_GATHER = _make_kernel()


def kernel(data, remain_idx):
    table = data.reshape(B * S, D)
    boff = jnp.arange(B, dtype=jnp.int32)[:, None] * S
    idx_full = jnp.concatenate(
        [boff, remain_idx.astype(jnp.int32) + 1 + boff], axis=1
    ).reshape(NROWS)
    return _GATHER(table, idx_full).reshape(B, OUT_S, D)


# speedup vs baseline: 2.1161x; 1.0724x over previous
import functools

import jax
import jax.numpy as jnp
from jax import lax
from jax.experimental import pallas as pl
from jax.experimental.pallas import tpu as pltpu
from jax.experimental.pallas import tpu_sc as plsc

B = 16
S = 4096
D = 1024
R = 2048
OUT_S = R + 1
NROWS = B * OUT_S
NC = 2
NS = 16
NW = NC * NS
RPW = 1024
XTRA = 8
CH = 32
NCH = RPW // CH


def _make_kernel():
    mesh = plsc.VectorSubcoreMesh(core_axis_name="c", subcore_axis_name="s")

    @functools.partial(
        pl.kernel,
        mesh=mesh,
        out_type=jax.ShapeDtypeStruct((NROWS, D), jnp.float32),
        scratch_types=[
            pltpu.VMEM((RPW + XTRA,), jnp.int32),
            pltpu.VMEM((CH, D), jnp.float32),
            pltpu.VMEM((CH, D), jnp.float32),
            pltpu.SemaphoreType.DMA,
            pltpu.SemaphoreType.DMA,
        ],
    )
    def gather_kernel(table_hbm, idx_hbm, out_hbm, idx_v, buf0, buf1, s0, s1):
        wid = lax.axis_index("s") * NC + lax.axis_index("c")
        base = wid * RPW

        pltpu.sync_copy(idx_hbm.at[pl.ds(base, RPW)], idx_v.at[pl.ds(0, RPW)])

        def start(c, buf, sem):
            pltpu.async_copy(
                table_hbm.at[idx_v.at[pl.ds(c * CH, CH)]], buf, sem
            )

        def wait(buf, sem):
            pltpu.make_async_copy(table_hbm.at[pl.ds(0, CH)], buf, sem).wait()

        def write(c, buf):
            pltpu.sync_copy(buf, out_hbm.at[pl.ds(base + c * CH, CH)])

        start(0, buf0, s0)

        def chunk_pair(i, carry):
            g = 2 * i
            start(g + 1, buf1, s1)
            wait(buf0, s0)
            write(g, buf0)

            @pl.when(g + 2 < NCH)
            def _():
                start(g + 2, buf0, s0)

            wait(buf1, s1)
            write(g + 1, buf1)
            return carry

        lax.fori_loop(0, NCH // 2, chunk_pair, 0)

        @pl.when(wid < 2)
        def _():
            xbase = NW * RPW + wid * XTRA
            pltpu.sync_copy(
                idx_hbm.at[pl.ds(xbase, XTRA)], idx_v.at[pl.ds(RPW, XTRA)]
            )
            pltpu.async_copy(
                table_hbm.at[idx_v.at[pl.ds(RPW, XTRA)]],
                buf0.at[pl.ds(0, XTRA)],
                s0,
            ).wait()
            pltpu.sync_copy(
                buf0.at[pl.ds(0, XTRA)], out_hbm.at[pl.ds(xbase, XTRA)]
            )

    return gather_kernel


_GATHER = _make_kernel()


def kernel(data, remain_idx):
    table = data.reshape(B * S, D)
    boff = jnp.arange(B, dtype=jnp.int32)[:, None] * S
    idx_full = jnp.concatenate(
        [boff, remain_idx.astype(jnp.int32) + 1 + boff], axis=1
    ).reshape(NROWS)
    return _GATHER(table, idx_full).reshape(B, OUT_S, D)


# speedup vs baseline: 2.9636x; 1.4005x over previous
import functools

import jax
import jax.numpy as jnp
from jax import lax
from jax.experimental import pallas as pl
from jax.experimental.pallas import tpu as pltpu
from jax.experimental.pallas import tpu_sc as plsc

B = 16
S = 4096
D = 1024
R = 2048
OUT_S = R + 1
NC = 2
NS = 16
NW = NC * NS
RPW = 1024
CH = 32
NCH = RPW // CH


def _make_kernel():
    mesh = plsc.VectorSubcoreMesh(core_axis_name="c", subcore_axis_name="s")

    @functools.partial(
        pl.kernel,
        mesh=mesh,
        out_type=jax.ShapeDtypeStruct((B, OUT_S, D), jnp.float32),
        scratch_types=[
            pltpu.VMEM((RPW + 128,), jnp.int32),
            pltpu.VMEM((CH, D), jnp.float32),
            pltpu.VMEM((CH, D), jnp.float32),
            pltpu.VMEM((1, D), jnp.float32),
            pltpu.SemaphoreType.DMA,
            pltpu.SemaphoreType.DMA,
        ],
    )
    def gather_kernel(
        table_hbm, idx_hbm, out_hbm, idx_v, buf0, buf1, tail_buf, s0, s1
    ):
        wid = lax.axis_index("s") * NC + lax.axis_index("c")
        b = wid // 2
        h = wid % 2
        row0 = h * RPW

        pltpu.sync_copy(idx_hbm.at[b, pl.ds(row0, RPW)], idx_v.at[pl.ds(0, RPW)])

        def start(c, buf, sem):
            pltpu.async_copy(
                table_hbm.at[idx_v.at[pl.ds(c * CH, CH)]], buf, sem
            )

        def wait(buf, sem):
            pltpu.make_async_copy(table_hbm.at[pl.ds(0, CH)], buf, sem).wait()

        def write(c, buf):
            pltpu.sync_copy(buf, out_hbm.at[b, pl.ds(row0 + c * CH, CH)])

        start(0, buf0, s0)

        def chunk_pair(i, carry):
            g = 2 * i
            start(g + 1, buf1, s1)
            wait(buf0, s0)
            write(g, buf0)

            @pl.when(g + 2 < NCH)
            def _():
                start(g + 2, buf0, s0)

            wait(buf1, s1)
            write(g + 1, buf1)
            return carry

        lax.fori_loop(0, NCH // 2, chunk_pair, 0)

        @pl.when(h == 1)
        def _():
            pltpu.sync_copy(
                idx_hbm.at[b, pl.ds(R, 128)], idx_v.at[pl.ds(RPW, 128)]
            )
            pltpu.async_copy(
                table_hbm.at[idx_v.at[pl.ds(RPW, 1)]], tail_buf, s0
            ).wait()
            pltpu.sync_copy(tail_buf, out_hbm.at[b, pl.ds(R, 1)])

    return gather_kernel


_GATHER = _make_kernel()


def kernel(data, remain_idx):
    table = data.reshape(B * S, D)
    boff = jnp.arange(B, dtype=jnp.int32)[:, None] * S
    idx_full = jnp.concatenate(
        [boff, remain_idx.astype(jnp.int32) + 1 + boff], axis=1
    )
    idx_full = jnp.pad(idx_full, ((0, 0), (0, 127)))
    return _GATHER(table, idx_full)


# speedup vs baseline: 5.9602x; 2.0111x over previous
import functools

import jax
import jax.numpy as jnp
from jax import lax
from jax.experimental import pallas as pl
from jax.experimental.pallas import tpu as pltpu
from jax.experimental.pallas import tpu_sc as plsc

B = 16
S = 4096
D = 1024
R = 2048
OUT_S = R + 1
NROWS = B * OUT_S
NC = 2
NS = 16
NW = NC * NS
RPW = 1024
TAIL = NROWS - NW * RPW
CH = 32
NCH = RPW // CH


def _make_kernel():
    mesh = plsc.VectorSubcoreMesh(core_axis_name="c", subcore_axis_name="s")

    @functools.partial(
        pl.kernel,
        mesh=mesh,
        out_type=jax.ShapeDtypeStruct((NROWS, D), jnp.float32),
        scratch_types=[
            pltpu.VMEM((RPW + TAIL,), jnp.int32),
            pltpu.VMEM((CH, D), jnp.float32),
            pltpu.VMEM((CH, D), jnp.float32),
            pltpu.VMEM((TAIL, D), jnp.float32),
            pltpu.SemaphoreType.DMA,
            pltpu.SemaphoreType.DMA,
        ],
    )
    def gather_kernel(
        table_hbm, idx_hbm, out_hbm, idx_v, buf0, buf1, tail_buf, s0, s1
    ):
        wid = lax.axis_index("s") * NC + lax.axis_index("c")
        base = wid * RPW

        pltpu.sync_copy(idx_hbm.at[pl.ds(base, RPW)], idx_v.at[pl.ds(0, RPW)])

        def start(c, buf, sem):
            pltpu.async_copy(
                table_hbm.at[idx_v.at[pl.ds(c * CH, CH)]], buf, sem
            )

        def wait(buf, sem):
            pltpu.make_async_copy(table_hbm.at[pl.ds(0, CH)], buf, sem).wait()

        def write(c, buf):
            pltpu.sync_copy(buf, out_hbm.at[pl.ds(base + c * CH, CH)])

        start(0, buf0, s0)

        def chunk_pair(i, carry):
            g = 2 * i
            start(g + 1, buf1, s1)
            wait(buf0, s0)
            write(g, buf0)

            @pl.when(g + 2 < NCH)
            def _():
                start(g + 2, buf0, s0)

            wait(buf1, s1)
            write(g + 1, buf1)
            return carry

        lax.fori_loop(0, NCH // 2, chunk_pair, 0)

        @pl.when(wid == NW - 1)
        def _():
            xbase = NW * RPW
            pltpu.sync_copy(
                idx_hbm.at[pl.ds(xbase, TAIL)], idx_v.at[pl.ds(RPW, TAIL)]
            )
            pltpu.async_copy(
                table_hbm.at[idx_v.at[pl.ds(RPW, TAIL)]], tail_buf, s0
            ).wait()
            pltpu.sync_copy(tail_buf, out_hbm.at[pl.ds(xbase, TAIL)])

    return gather_kernel


_GATHER = _make_kernel()


def kernel(data, remain_idx):
    table = data.reshape(B * S, D)
    boff = jnp.arange(B, dtype=jnp.int32) * S
    idx_t = jnp.concatenate(
        [boff[None, :], remain_idx.astype(jnp.int32).T + 1 + boff[None, :]],
        axis=0,
    ).reshape(NROWS)
    out_flat = _GATHER(table, idx_t)
    return out_flat.reshape(OUT_S, B, D).transpose(1, 0, 2)
